# baseline (speedup 1.0000x reference)
import functools

import jax
import jax.numpy as jnp
import numpy as np
from jax import lax
from jax.experimental import pallas as pl
from jax.experimental.pallas import tpu as pltpu
from jax.experimental.pallas import tpu_sc as plsc

B = 16384
L = 50
D = 128
A = 1000
H = 64
AP = 1024
R = L * AP
AOUT = 1024

NC = 2
NS = 16
NW = NC * NS
NSPLIT = 2
BS = B // NSPLIT
SPW = BS // NW
CHUNK = 16
ROWS = CHUNK * L
NCHUNKS = SPW // CHUNK


def _rne_bf16_bits(t):
    u = lax.bitcast_convert_type(t, jnp.uint32)
    return (u + 0x7FFF + ((u >> 16) & 1)) >> 16


KL = 5


def _table_body(e_ref, w_ref, t_ref):
    for k in range(KL):
        t = jnp.dot(e_ref[:], w_ref[k], preferred_element_type=jnp.float32)
        ra = _rne_bf16_bits(t[:, : H // 2])
        rb = _rne_bf16_bits(t[:, H // 2 :])
        t_ref[k] = lax.bitcast_convert_type(ra | (rb << 16), jnp.int32)


def _build_table(embed_p, w1t):
    return pl.pallas_call(
        _table_body,
        grid=(L // KL,),
        in_specs=[
            pl.BlockSpec((AP, D), lambda g: (0, 0)),
            pl.BlockSpec((KL, D, H), lambda g: (g, 0, 0)),
        ],
        out_specs=pl.BlockSpec((KL, AP, H // 2), lambda g: (g, 0, 0)),
        out_shape=jax.ShapeDtypeStruct((L, AP, H // 2), jnp.int32),
    )(embed_p, w1t)


def _bag_body(table_hbm, idx_hbm, out_hbm, idx_v, rows0, rows1, h1_v, sem0, sem1):
    wid = lax.axis_index("s") * NC + lax.axis_index("c")
    base = wid * SPW
    pltpu.sync_copy(idx_hbm.at[pl.ds(base * L, SPW * L)], idx_v)

    def gather(c, rows, sem):
        return pltpu.make_async_copy(
            table_hbm.at[idx_v.at[pl.ds(c * ROWS, ROWS)]], rows, sem
        )

    hi_mask = jnp.full((16,), -65536, dtype=jnp.int32)

    lanes = lax.iota(jnp.int32, 16)
    rowidx = [lanes + j * 32 + e * 16 for j in range(H // 32) for e in (0, 1)]

    def accum(c, rows):
        def samp_body(s, carry):
            r0 = s * L
            col = jnp.full((16,), s, dtype=jnp.int32)
            for j in range(H // 32):
                acc_lo = jnp.zeros((16,), jnp.float32)
                acc_hi = jnp.zeros((16,), jnp.float32)
                for l in range(L):
                    w = rows[r0 + l, pl.ds(j * 16, 16)]
                    acc_lo = acc_lo + plsc.bitcast(
                        lax.shift_left(w, 16), jnp.float32
                    )
                    acc_hi = acc_hi + plsc.bitcast(
                        lax.bitwise_and(w, hi_mask), jnp.float32
                    )
                plsc.store_scatter(h1_v, [rowidx[2 * j], col], acc_lo)
                plsc.store_scatter(h1_v, [rowidx[2 * j + 1], col], acc_hi)
            return carry

        lax.fori_loop(0, CHUNK, samp_body, 0)
        pltpu.sync_copy(h1_v, out_hbm.at[:, pl.ds(base + c * CHUNK, CHUNK)])

    last = NCHUNKS - 1
    gather(0, rows0, sem0).start()

    def pair_body(k, carry):
        c0 = 2 * k
        gather(c0, rows0, sem0).wait()
        gather(c0 + 1, rows1, sem1).start()
        accum(c0, rows0)
        gather(c0 + 1, rows1, sem1).wait()
        gather(lax.min(c0 + 2, last), rows0, sem0).start()
        accum(c0 + 1, rows1)
        return carry

    lax.fori_loop(0, NCHUNKS // 2, pair_body, 0)
    gather(last, rows0, sem0).wait()


@functools.cache
def _bag():
    return pl.kernel(
        _bag_body,
        mesh=plsc.VectorSubcoreMesh(core_axis_name="c", subcore_axis_name="s"),
        compiler_params=pltpu.CompilerParams(
            use_tc_tiling_on_sc=False, needs_layout_passes=False
        ),
        out_type=jax.ShapeDtypeStruct((H, BS), jnp.float32),
        scratch_types=[
            pltpu.VMEM((SPW * L,), jnp.int32),
            pltpu.VMEM((ROWS, H // 2), jnp.int32),
            pltpu.VMEM((ROWS, H // 2), jnp.int32),
            pltpu.VMEM((H, CHUNK), jnp.float32),
            pltpu.SemaphoreType.DMA,
            pltpu.SemaphoreType.DMA,
        ],
    )


BT = 2048


def _mlp2_body(h_ref, b1_ref, w2_ref, b2_ref, o_ref):
    h = jnp.maximum(h_ref[:] + b1_ref[:], 0.0)
    o_ref[:] = (
        jnp.dot(w2_ref[:], h, preferred_element_type=jnp.float32) + b2_ref[:]
    )


def _mlp2_body_alias(prev_ref, h_ref, b1_ref, w2_ref, b2_ref, o_ref):
    del prev_ref
    _mlp2_body(h_ref, b1_ref, w2_ref, b2_ref, o_ref)


def _mlp2_part(h1t, b1c, w2c, b2c, part, prev=None):
    ntile = BS // BT
    common = dict(
        grid=(ntile,),
        out_specs=pl.BlockSpec((A, BT), lambda i, p=part: (0, i + p * ntile)),
        out_shape=jax.ShapeDtypeStruct((A, B), jnp.float32),
    )
    hspecs = [
        pl.BlockSpec((H, BT), lambda i: (0, i)),
        pl.BlockSpec((H, 1), lambda i: (0, 0)),
        pl.BlockSpec((A, H), lambda i: (0, 0)),
        pl.BlockSpec((A, 1), lambda i: (0, 0)),
    ]
    if prev is None:
        return pl.pallas_call(_mlp2_body, in_specs=hspecs, **common)(
            h1t, b1c, w2c, b2c
        )
    return pl.pallas_call(
        _mlp2_body_alias,
        in_specs=[pl.BlockSpec(memory_space=pltpu.MemorySpace.HBM)] + hspecs,
        input_output_aliases={0: 0},
        **common,
    )(prev, h1t, b1c, w2c, b2c)


_PERM = np.asarray(
    [j * 16 + 32 * e + i for j in range(H // 32) for e in (0, 1) for i in range(16)],
    dtype=np.int32,
)


def kernel(x, embed, W1, b1, W2, b2):
    embed_p = jnp.zeros((AP, D), jnp.float32).at[: A + 1, :].set(embed)
    w1t = W1.reshape(H, L, D).transpose(1, 2, 0)
    table_i32 = _build_table(embed_p, w1t).reshape(R, H // 2)

    flat_idx = (
        x.astype(jnp.int32) + (jnp.arange(L, dtype=jnp.int32) * AP)[None, :]
    ).reshape(-1)

    b1c = b1[_PERM].reshape(H, 1)
    w2c = W2[:, _PERM]
    b2c = b2.reshape(A, 1)

    h1ts = [
        _bag()(table_i32, flat_idx[p * BS * L : (p + 1) * BS * L])
        for p in range(NSPLIT)
    ]
    out_t = None
    for p in range(NSPLIT):
        out_t = _mlp2_part(h1ts[p], b1c, w2c, b2c, p, prev=out_t)
    return out_t.T


# speedup vs baseline: 1.1715x; 1.1715x over previous
import functools

import jax
import jax.numpy as jnp
import numpy as np
from jax import lax
from jax.experimental import pallas as pl
from jax.experimental.pallas import tpu as pltpu
from jax.experimental.pallas import tpu_sc as plsc

B = 16384
L = 50
D = 128
A = 1000
H = 64
AP = 1024
R = L * AP
AOUT = 1024

NC = 2
NS = 16
NW = NC * NS
NSPLIT = 1
BS = B // NSPLIT
SPW = BS // NW
CHUNK = 16
ROWS = CHUNK * L
NCHUNKS = SPW // CHUNK


def _rne_bf16_bits(t):
    u = lax.bitcast_convert_type(t, jnp.uint32)
    return (u + 0x7FFF + ((u >> 16) & 1)) >> 16


KL = 5
QR = AP // 4


def _table_body(e0, e1, e2, e3, w_ref, t_ref):
    es = (e0, e1, e2, e3)
    for k in range(KL):
        parts = []
        for c in range(4):
            t = jnp.dot(
                es[c][:], w_ref[k], preferred_element_type=jnp.float32
            )
            ra = _rne_bf16_bits(t[:, : H // 2])
            rb = _rne_bf16_bits(t[:, H // 2 :])
            parts.append(lax.bitcast_convert_type(ra | (rb << 16), jnp.int32))
        t_ref[pl.ds(k * QR, QR), :] = jnp.concatenate(parts, axis=1)


def _build_table(e4, w1t):
    espec = pl.BlockSpec((QR, D), lambda g: (0, 0))
    return pl.pallas_call(
        _table_body,
        grid=(L // KL,),
        in_specs=[
            espec,
            espec,
            espec,
            espec,
            pl.BlockSpec((KL, D, H), lambda g: (g, 0, 0)),
        ],
        out_specs=pl.BlockSpec((KL * QR, 128), lambda g: (g, 0)),
        out_shape=jax.ShapeDtypeStruct((R * (H // 2) // 128, 128), jnp.int32),
    )(*e4, w1t)


def _bag_body(table_hbm, idx_hbm, out_hbm, idx_v, rows0, rows1, h1_v, sem0, sem1):
    wid = lax.axis_index("s") * NC + lax.axis_index("c")
    base = wid * SPW
    pltpu.sync_copy(idx_hbm.at[pl.ds(base * L, SPW * L)], idx_v)

    def gather(c, rows, sem):
        return pltpu.make_async_copy(
            table_hbm.at[idx_v.at[pl.ds(c * ROWS, ROWS)]], rows, sem
        )

    hi_mask = jnp.full((16,), -65536, dtype=jnp.int32)

    lanes = lax.iota(jnp.int32, 16)
    rowidx = [lanes + j * 32 + e * 16 for j in range(H // 32) for e in (0, 1)]

    def accum(c, rows):
        def samp_body(s, carry):
            r0 = s * L
            col = jnp.full((16,), s, dtype=jnp.int32)
            for j in range(H // 32):
                acc_lo = jnp.zeros((16,), jnp.float32)
                acc_hi = jnp.zeros((16,), jnp.float32)
                for l in range(L):
                    w = rows[r0 + l, pl.ds(j * 16, 16)]
                    acc_lo = acc_lo + plsc.bitcast(
                        lax.shift_left(w, 16), jnp.float32
                    )
                    acc_hi = acc_hi + plsc.bitcast(
                        lax.bitwise_and(w, hi_mask), jnp.float32
                    )
                plsc.store_scatter(h1_v, [rowidx[2 * j], col], acc_lo)
                plsc.store_scatter(h1_v, [rowidx[2 * j + 1], col], acc_hi)
            return carry

        lax.fori_loop(0, CHUNK, samp_body, 0)
        pltpu.sync_copy(h1_v, out_hbm.at[:, pl.ds(base + c * CHUNK, CHUNK)])

    last = NCHUNKS - 1
    gather(0, rows0, sem0).start()

    def pair_body(k, carry):
        c0 = 2 * k
        gather(c0, rows0, sem0).wait()
        gather(c0 + 1, rows1, sem1).start()
        accum(c0, rows0)
        gather(c0 + 1, rows1, sem1).wait()
        gather(lax.min(c0 + 2, last), rows0, sem0).start()
        accum(c0 + 1, rows1)
        return carry

    lax.fori_loop(0, NCHUNKS // 2, pair_body, 0)
    gather(last, rows0, sem0).wait()


@functools.cache
def _bag():
    return pl.kernel(
        _bag_body,
        mesh=plsc.VectorSubcoreMesh(core_axis_name="c", subcore_axis_name="s"),
        compiler_params=pltpu.CompilerParams(
            use_tc_tiling_on_sc=False, needs_layout_passes=False
        ),
        out_type=jax.ShapeDtypeStruct((H, BS), jnp.float32),
        scratch_types=[
            pltpu.VMEM((SPW * L,), jnp.int32),
            pltpu.VMEM((ROWS, H // 2), jnp.int32),
            pltpu.VMEM((ROWS, H // 2), jnp.int32),
            pltpu.VMEM((H, CHUNK), jnp.float32),
            pltpu.SemaphoreType.DMA,
            pltpu.SemaphoreType.DMA,
        ],
    )


BT = 2048


def _mlp2_body(h_ref, b1_ref, w2_ref, b2_ref, o_ref):
    h = jnp.maximum(h_ref[:] + b1_ref[:], 0.0)
    o_ref[:] = (
        jnp.dot(w2_ref[:], h, preferred_element_type=jnp.float32) + b2_ref[:]
    )


def _mlp2_body_alias(prev_ref, h_ref, b1_ref, w2_ref, b2_ref, o_ref):
    del prev_ref
    _mlp2_body(h_ref, b1_ref, w2_ref, b2_ref, o_ref)


def _mlp2_part(h1t, b1c, w2c, b2c, part, prev=None):
    ntile = BS // BT
    common = dict(
        grid=(ntile,),
        out_specs=pl.BlockSpec((A, BT), lambda i, p=part: (0, i + p * ntile)),
        out_shape=jax.ShapeDtypeStruct((A, B), jnp.float32),
    )
    hspecs = [
        pl.BlockSpec((H, BT), lambda i: (0, i)),
        pl.BlockSpec((H, 1), lambda i: (0, 0)),
        pl.BlockSpec((A, H), lambda i: (0, 0)),
        pl.BlockSpec((A, 1), lambda i: (0, 0)),
    ]
    if prev is None:
        return pl.pallas_call(_mlp2_body, in_specs=hspecs, **common)(
            h1t, b1c, w2c, b2c
        )
    return pl.pallas_call(
        _mlp2_body_alias,
        in_specs=[pl.BlockSpec(memory_space=pltpu.MemorySpace.HBM)] + hspecs,
        input_output_aliases={0: 0},
        **common,
    )(prev, h1t, b1c, w2c, b2c)


_PERM = np.asarray(
    [j * 16 + 32 * e + i for j in range(H // 32) for e in (0, 1) for i in range(16)],
    dtype=np.int32,
)


def kernel(x, embed, W1, b1, W2, b2):
    embed_p = jnp.zeros((AP, D), jnp.float32).at[: A + 1, :].set(embed)
    e4 = [embed_p[c::4] for c in range(4)]
    w1t = W1.reshape(H, L, D).transpose(1, 2, 0)
    table_i32 = _build_table(e4, w1t).reshape(R, H // 2)

    flat_idx = (
        x.astype(jnp.int32) + (jnp.arange(L, dtype=jnp.int32) * AP)[None, :]
    ).reshape(-1)

    b1c = b1[_PERM].reshape(H, 1)
    w2c = W2[:, _PERM]
    b2c = b2.reshape(A, 1)

    h1ts = [
        _bag()(table_i32, flat_idx[p * BS * L : (p + 1) * BS * L])
        for p in range(NSPLIT)
    ]
    out_t = None
    for p in range(NSPLIT):
        out_t = _mlp2_part(h1ts[p], b1c, w2c, b2c, p, prev=out_t)
    return out_t.T
